# baseline (speedup 1.0000x reference)
import functools

import jax
import jax.numpy as jnp
from jax import lax
from jax.experimental import pallas as pl
from jax.experimental.pallas import tpu as pltpu
from jax.experimental.pallas import tpu_sc as plsc

NC = 2
NS = 16
NW = NC * NS
CH = 1000


_SC_PARAMS = pltpu.CompilerParams(use_tc_tiling_on_sc=False)


def _mesh():
    return plsc.VectorSubcoreMesh(core_axis_name="c", subcore_axis_name="s",
                                  num_cores=NC, num_subcores=NS)


def _sc_gather(table, idx):
    n, f = table.shape
    e = idx.shape[0]
    per_w = e // NW
    nch = per_w // CH

    @functools.partial(
        pl.kernel,
        out_type=jax.ShapeDtypeStruct((e, f), jnp.float32),
        mesh=_mesh(),
        compiler_params=_SC_PARAMS,
        scratch_types=[
            pltpu.VMEM((CH,), jnp.int32),
            pltpu.VMEM((CH, f), jnp.float32),
            pltpu.SemaphoreType.DMA,
        ],
    )
    def gk(table_hbm, idx_hbm, out_hbm, idx_v, rows_v, sem):
        wid = lax.axis_index("s") * NC + lax.axis_index("c")
        base = wid * per_w
        for k in range(nch):
            off = base + k * CH
            pltpu.sync_copy(idx_hbm.at[pl.ds(off, CH)], idx_v)
            pltpu.async_copy(table_hbm.at[idx_v], rows_v, sem).wait()
            pltpu.sync_copy(rows_v, out_hbm.at[pl.ds(off, CH)])

    return gk(table, idx)


def _sc_scatter_update(msg48, dst, z):
    e = dst.shape[0]
    n = z.shape[0]
    nl = n // NC
    ntr = (nl + NS) // NS
    npad = ntr * NS
    per_t = e // NS
    ch = 1200
    nch = per_t // ch
    tail = per_t - nch * ch
    last = nl - (NS - 1) * ntr
    zeros = jnp.zeros((npad, 48), jnp.float32)

    @functools.partial(
        pl.kernel,
        out_type=jax.ShapeDtypeStruct((n, 32), jnp.float32),
        mesh=_mesh(),
        compiler_params=_SC_PARAMS,
        scratch_types=[
            pltpu.VMEM((ch, 48), jnp.float32),
            pltpu.VMEM((ch,), jnp.int32),
            pltpu.VMEM((tail, 48), jnp.float32),
            pltpu.VMEM((tail,), jnp.int32),
            pltpu.VMEM((ntr, 48), jnp.float32),
            pltpu.VMEM((ntr, 32), jnp.float32),
            pltpu.VMEM_SHARED((npad, 48), jnp.float32),
        ],
    )
    def sk(msg_hbm, dst_hbm, z_hbm, zeros_hbm, out_hbm,
           msg_v, idx_v, msgt_v, idxt_v, acc_t, z_t, acc_sh):
        c = lax.axis_index("c")
        s = lax.axis_index("s")
        lo = c * nl
        pltpu.sync_copy(zeros_hbm.at[pl.ds(s * ntr, ntr)],
                        acc_sh.at[pl.ds(s * ntr, ntr)])
        plsc.subcore_barrier()

        def remap(idx_ref, nvec):
            def one(v, carry):
                vec = idx_ref[pl.ds(v * 16, 16)]
                loc = vec - lo
                ok = (loc >= 0) & (loc < nl)
                trash = nl + (vec & 7)
                idx_ref[pl.ds(v * 16, 16)] = jnp.where(ok, loc, trash)
                return carry
            lax.fori_loop(0, nvec, one, 0)

        base = s * per_t
        for k in range(nch):
            off = base + k * ch
            pltpu.sync_copy(dst_hbm.at[pl.ds(off, ch)], idx_v)
            pltpu.sync_copy(msg_hbm.at[pl.ds(off, ch)], msg_v)
            remap(idx_v, ch // 16)
            pltpu.sync_copy(msg_v, acc_sh.at[idx_v], add=True)
        off = base + nch * ch
        pltpu.sync_copy(dst_hbm.at[pl.ds(off, tail)], idxt_v)
        pltpu.sync_copy(msg_hbm.at[pl.ds(off, tail)], msgt_v)
        remap(idxt_v, tail // 16)
        pltpu.sync_copy(msgt_v, acc_sh.at[idxt_v], add=True)
        plsc.subcore_barrier()

        row0 = s * ntr
        grow = lo + row0
        pltpu.sync_copy(acc_sh.at[pl.ds(row0, ntr)], acc_t)

        def upd(r, carry):
            degv = acc_t[r, pl.ds(32, 16)]
            dinv = 1.0 / jnp.maximum(degv, 1.0)
            for h in range(2):
                a = acc_t[r, pl.ds(h * 16, 16)]
                zz = z_t[r, pl.ds(h * 16, 16)]
                z_t[r, pl.ds(h * 16, 16)] = jnp.maximum(a * dinv + zz, 0.0)
            return carry

        @pl.when(s < NS - 1)
        def _():
            pltpu.sync_copy(z_hbm.at[pl.ds(grow, ntr)], z_t)
            lax.fori_loop(0, ntr, upd, 0)
            pltpu.sync_copy(z_t, out_hbm.at[pl.ds(grow, ntr)])

        @pl.when(s == NS - 1)
        def _():
            pltpu.sync_copy(z_hbm.at[pl.ds(grow, last)], z_t.at[pl.ds(0, last)])
            lax.fori_loop(0, last, upd, 0)
            pltpu.sync_copy(z_t.at[pl.ds(0, last)], out_hbm.at[pl.ds(grow, last)])

    return sk(msg48, dst, z, zeros)


def _tc_node_proj_z(h, w0, b0, wroot, bconv):
    n, d = h.shape
    f = w0.shape[1]
    bn = 2000

    def body(h_ref, w_ref, b_ref, wr_ref, bc_ref, o_ref, z_ref):
        acc = jnp.dot(h_ref[...], w_ref[...], preferred_element_type=jnp.float32)
        o = jnp.maximum(acc + b_ref[...], 0.0)
        o_ref[...] = o
        z_ref[...] = jnp.dot(o, wr_ref[...],
                             preferred_element_type=jnp.float32) + bc_ref[...]

    return pl.pallas_call(
        body,
        grid=(n // bn,),
        in_specs=[
            pl.BlockSpec((bn, d), lambda i: (i, 0)),
            pl.BlockSpec((d, f), lambda i: (0, 0)),
            pl.BlockSpec((1, f), lambda i: (0, 0)),
            pl.BlockSpec((f, f), lambda i: (0, 0)),
            pl.BlockSpec((1, f), lambda i: (0, 0)),
        ],
        out_specs=[
            pl.BlockSpec((bn, f), lambda i: (i, 0)),
            pl.BlockSpec((bn, f), lambda i: (i, 0)),
        ],
        out_shape=[
            jax.ShapeDtypeStruct((n, f), jnp.float32),
            jax.ShapeDtypeStruct((n, f), jnp.float32),
        ],
    )(h, w0, b0.reshape(1, f), wroot, bconv.reshape(1, f))


def _tc_z(out, wroot, bconv):
    n, f = out.shape
    bn = 2000

    def body(o_ref, wr_ref, bc_ref, z_ref):
        z_ref[...] = jnp.dot(o_ref[...], wr_ref[...],
                             preferred_element_type=jnp.float32) + bc_ref[...]

    return pl.pallas_call(
        body,
        grid=(n // bn,),
        in_specs=[
            pl.BlockSpec((bn, f), lambda i: (i, 0)),
            pl.BlockSpec((f, f), lambda i: (0, 0)),
            pl.BlockSpec((1, f), lambda i: (0, 0)),
        ],
        out_specs=pl.BlockSpec((bn, f), lambda i: (i, 0)),
        out_shape=jax.ShapeDtypeStruct((n, f), jnp.float32),
    )(out, wroot, bconv.reshape(1, f))


def _tc_edge_mlp(edge_attr, ws, bs, w1, b1):
    e, ng = edge_attr.shape
    k3 = ws.shape[1]
    hid = w1.shape[1]
    be = 2000

    def body(a_ref, ws_ref, bs_ref, w1_ref, b1_ref, o_ref):
        ea = jnp.dot(a_ref[...], ws_ref[...], preferred_element_type=jnp.float32)
        ea = jnp.maximum(ea + bs_ref[...], 0.0)
        t = jnp.dot(ea, w1_ref[...], preferred_element_type=jnp.float32)
        o_ref[...] = jnp.maximum(t + b1_ref[...], 0.0).astype(jnp.bfloat16)

    return pl.pallas_call(
        body,
        grid=(e // be,),
        in_specs=[
            pl.BlockSpec((be, ng), lambda i: (i, 0)),
            pl.BlockSpec((ng, k3), lambda i: (0, 0)),
            pl.BlockSpec((1, k3), lambda i: (0, 0)),
            pl.BlockSpec((k3, hid), lambda i: (0, 0)),
            pl.BlockSpec((1, hid), lambda i: (0, 0)),
        ],
        out_specs=pl.BlockSpec((be, hid), lambda i: (i, 0)),
        out_shape=jax.ShapeDtypeStruct((e, hid), jnp.bfloat16),
    )(edge_attr, ws, bs.reshape(1, k3), w1, b1.reshape(1, hid))


def _tc_msg(t, xg, w2p, b2p, gmat):
    e, hid = t.shape
    nf = xg.shape[1]
    kk = nf * nf
    be = 4000

    def body(t_ref, x_ref, w2_ref, b2_ref, g_ref, o_ref):
        we = jnp.dot(t_ref[...], w2_ref[...], preferred_element_type=jnp.float32)
        we = (we + b2_ref[...]).astype(jnp.bfloat16)
        xt = jnp.tile(x_ref[...].astype(jnp.bfloat16), (1, nf))
        m = xt * we
        res = jnp.dot(m, g_ref[...], preferred_element_type=jnp.float32)
        o_ref[...] = jnp.concatenate(
            [res, jnp.ones((be, 16), jnp.float32)], axis=1)

    return pl.pallas_call(
        body,
        grid=(e // be,),
        in_specs=[
            pl.BlockSpec((be, hid), lambda i: (i, 0)),
            pl.BlockSpec((be, nf), lambda i: (i, 0)),
            pl.BlockSpec((hid, kk), lambda i: (0, 0)),
            pl.BlockSpec((1, kk), lambda i: (0, 0)),
            pl.BlockSpec((kk, nf), lambda i: (0, 0)),
        ],
        out_specs=pl.BlockSpec((be, nf + 16), lambda i: (i, 0)),
        out_shape=jax.ShapeDtypeStruct((e, nf + 16), jnp.float32),
    )(t, xg, w2p, b2p.reshape(1, kk), gmat)


def kernel(h, edge_index, edge_weight, edge_attr, data,
           W0, b0, Ws, bs, W1, b1, W2, b2, Wroot, bconv):
    nf = W0.shape[1]
    src = edge_index[0].astype(jnp.int32)
    dst = edge_index[1].astype(jnp.int32)

    hid = W1.shape[1]
    w2p = W2.reshape(hid, nf, nf).transpose(0, 2, 1).reshape(hid, nf * nf)
    w2p = w2p.astype(jnp.bfloat16)
    b2p = b2.reshape(nf, nf).T.reshape(nf * nf)
    gmat = jnp.repeat(jnp.eye(nf, dtype=jnp.bfloat16), nf, axis=0)

    out0, z0 = _tc_node_proj_z(h, W0, b0, Wroot, bconv)
    t = _tc_edge_mlp(edge_attr, Ws, bs, W1, b1)

    xg1 = _sc_gather(out0, src)
    msg1 = _tc_msg(t, xg1, w2p, b2p, gmat)
    out1 = _sc_scatter_update(msg1, dst, z0)

    z1 = _tc_z(out1, Wroot, bconv)
    xg2 = _sc_gather(out1, src)
    msg2 = _tc_msg(t, xg2, w2p, b2p, gmat)
    out2 = _sc_scatter_update(msg2, dst, z1)
    return out2


# speedup vs baseline: 1.0699x; 1.0699x over previous
import functools

import jax
import jax.numpy as jnp
from jax import lax
from jax.experimental import pallas as pl
from jax.experimental.pallas import tpu as pltpu
from jax.experimental.pallas import tpu_sc as plsc

NC = 2
NS = 16
NW = NC * NS
CH = 1000


_SC_PARAMS = pltpu.CompilerParams(use_tc_tiling_on_sc=False)


def _mesh():
    return plsc.VectorSubcoreMesh(core_axis_name="c", subcore_axis_name="s",
                                  num_cores=NC, num_subcores=NS)


def _sc_gather(table, idx):
    n, f = table.shape
    e = idx.shape[0]
    per_w = e // NW
    nch = per_w // CH

    @functools.partial(
        pl.kernel,
        out_type=jax.ShapeDtypeStruct((e, 128), jnp.float32),
        mesh=_mesh(),
        compiler_params=_SC_PARAMS,
        scratch_types=[
            pltpu.VMEM((CH,), jnp.int32),
            pltpu.VMEM((CH, f), jnp.float32),
            pltpu.SemaphoreType.DMA,
        ],
    )
    def gk(table_hbm, idx_hbm, out_hbm, idx_v, rows_v, sem):
        wid = lax.axis_index("s") * NC + lax.axis_index("c")
        base = wid * per_w
        for k in range(nch):
            off = base + k * CH
            pltpu.sync_copy(idx_hbm.at[pl.ds(off, CH)], idx_v)
            pltpu.async_copy(table_hbm.at[idx_v], rows_v, sem).wait()
            pltpu.sync_copy(rows_v, out_hbm.at[pl.ds(off, CH), pl.ds(0, f)])

    return gk(table, idx)


def _sc_scatter_update(msg48, dst, z):
    e = dst.shape[0]
    n = z.shape[0]
    nl = n // NC
    ntr = (nl + NS) // NS
    npad = ntr * NS
    per_t = e // NS
    ch = 1200
    nch = per_t // ch
    tail = per_t - nch * ch
    last = nl - (NS - 1) * ntr
    zeros = jnp.zeros((npad, 48), jnp.float32)

    @functools.partial(
        pl.kernel,
        out_type=jax.ShapeDtypeStruct((n, 32), jnp.float32),
        mesh=_mesh(),
        compiler_params=_SC_PARAMS,
        scratch_types=[
            pltpu.VMEM((ch, 48), jnp.float32),
            pltpu.VMEM((ch,), jnp.int32),
            pltpu.VMEM((tail, 48), jnp.float32),
            pltpu.VMEM((tail,), jnp.int32),
            pltpu.VMEM((ntr, 48), jnp.float32),
            pltpu.VMEM((ntr, 32), jnp.float32),
            pltpu.VMEM_SHARED((npad, 48), jnp.float32),
        ],
    )
    def sk(msg_hbm, dst_hbm, z_hbm, zeros_hbm, out_hbm,
           msg_v, idx_v, msgt_v, idxt_v, acc_t, z_t, acc_sh):
        c = lax.axis_index("c")
        s = lax.axis_index("s")
        lo = c * nl
        pltpu.sync_copy(zeros_hbm.at[pl.ds(s * ntr, ntr)],
                        acc_sh.at[pl.ds(s * ntr, ntr)])
        plsc.subcore_barrier()

        def remap(idx_ref, nvec):
            def one(v, carry):
                vec = idx_ref[pl.ds(v * 16, 16)]
                loc = vec - lo
                ok = (loc >= 0) & (loc < nl)
                trash = nl + (vec & 7)
                idx_ref[pl.ds(v * 16, 16)] = jnp.where(ok, loc, trash)
                return carry
            lax.fori_loop(0, nvec, one, 0)

        base = s * per_t
        for k in range(nch):
            off = base + k * ch
            pltpu.sync_copy(dst_hbm.at[pl.ds(off, ch)], idx_v)
            pltpu.sync_copy(msg_hbm.at[pl.ds(off, ch)], msg_v)
            remap(idx_v, ch // 16)
            pltpu.sync_copy(msg_v, acc_sh.at[idx_v], add=True)
        off = base + nch * ch
        pltpu.sync_copy(dst_hbm.at[pl.ds(off, tail)], idxt_v)
        pltpu.sync_copy(msg_hbm.at[pl.ds(off, tail)], msgt_v)
        remap(idxt_v, tail // 16)
        pltpu.sync_copy(msgt_v, acc_sh.at[idxt_v], add=True)
        plsc.subcore_barrier()

        row0 = s * ntr
        grow = lo + row0
        pltpu.sync_copy(acc_sh.at[pl.ds(row0, ntr)], acc_t)

        def upd(r, carry):
            degv = acc_t[r, pl.ds(32, 16)]
            dinv = 1.0 / jnp.maximum(degv, 1.0)
            for h in range(2):
                a = acc_t[r, pl.ds(h * 16, 16)]
                zz = z_t[r, pl.ds(h * 16, 16)]
                z_t[r, pl.ds(h * 16, 16)] = jnp.maximum(a * dinv + zz, 0.0)
            return carry

        @pl.when(s < NS - 1)
        def _():
            pltpu.sync_copy(z_hbm.at[pl.ds(grow, ntr)], z_t)
            lax.fori_loop(0, ntr, upd, 0)
            pltpu.sync_copy(z_t, out_hbm.at[pl.ds(grow, ntr)])

        @pl.when(s == NS - 1)
        def _():
            pltpu.sync_copy(z_hbm.at[pl.ds(grow, last)], z_t.at[pl.ds(0, last)])
            lax.fori_loop(0, last, upd, 0)
            pltpu.sync_copy(z_t.at[pl.ds(0, last)], out_hbm.at[pl.ds(grow, last)])

    return sk(msg48, dst, z, zeros)


def _tc_node_proj_z(h, w0, b0, wroot, bconv):
    n, d = h.shape
    f = w0.shape[1]
    bn = 2000

    def body(h_ref, w_ref, b_ref, wr_ref, bc_ref, o_ref, z_ref):
        acc = jnp.dot(h_ref[...], w_ref[...], preferred_element_type=jnp.float32)
        o = jnp.maximum(acc + b_ref[...], 0.0)
        o_ref[...] = o
        z_ref[...] = jnp.dot(o, wr_ref[...],
                             preferred_element_type=jnp.float32) + bc_ref[...]

    return pl.pallas_call(
        body,
        grid=(n // bn,),
        in_specs=[
            pl.BlockSpec((bn, d), lambda i: (i, 0)),
            pl.BlockSpec((d, f), lambda i: (0, 0)),
            pl.BlockSpec((1, f), lambda i: (0, 0)),
            pl.BlockSpec((f, f), lambda i: (0, 0)),
            pl.BlockSpec((1, f), lambda i: (0, 0)),
        ],
        out_specs=[
            pl.BlockSpec((bn, f), lambda i: (i, 0)),
            pl.BlockSpec((bn, f), lambda i: (i, 0)),
        ],
        out_shape=[
            jax.ShapeDtypeStruct((n, f), jnp.float32),
            jax.ShapeDtypeStruct((n, f), jnp.float32),
        ],
    )(h, w0, b0.reshape(1, f), wroot, bconv.reshape(1, f))


def _tc_z(out, wroot, bconv):
    n, f = out.shape
    bn = 2000

    def body(o_ref, wr_ref, bc_ref, z_ref):
        z_ref[...] = jnp.dot(o_ref[...], wr_ref[...],
                             preferred_element_type=jnp.float32) + bc_ref[...]

    return pl.pallas_call(
        body,
        grid=(n // bn,),
        in_specs=[
            pl.BlockSpec((bn, f), lambda i: (i, 0)),
            pl.BlockSpec((f, f), lambda i: (0, 0)),
            pl.BlockSpec((1, f), lambda i: (0, 0)),
        ],
        out_specs=pl.BlockSpec((bn, f), lambda i: (i, 0)),
        out_shape=jax.ShapeDtypeStruct((n, f), jnp.float32),
    )(out, wroot, bconv.reshape(1, f))


def _tc_edge_mlp(edge_attr, ws, bs, w1, b1):
    e, ng = edge_attr.shape
    k3 = ws.shape[1]
    hid = w1.shape[1]
    pk = 128 // ng
    ap = edge_attr.reshape(e // pk, pk * ng)
    wsb = jnp.kron(jnp.eye(pk, dtype=jnp.float32), ws)
    bsb = jnp.tile(bs, pk).reshape(1, pk * k3)
    w1b = jnp.kron(jnp.eye(pk, dtype=jnp.float32), w1)
    b1b = jnp.tile(b1, pk).reshape(1, pk * hid)
    br = 400

    def body(a_ref, ws_ref, bs_ref, w1_ref, b1_ref, o_ref):
        ea = jnp.dot(a_ref[...], ws_ref[...], preferred_element_type=jnp.float32)
        ea = jnp.maximum(ea + bs_ref[...], 0.0)
        t = jnp.dot(ea, w1_ref[...], preferred_element_type=jnp.float32)
        o_ref[...] = jnp.maximum(t + b1_ref[...], 0.0).astype(jnp.bfloat16)

    tp = pl.pallas_call(
        body,
        grid=(e // pk // br,),
        in_specs=[
            pl.BlockSpec((br, pk * ng), lambda i: (i, 0)),
            pl.BlockSpec((pk * ng, pk * k3), lambda i: (0, 0)),
            pl.BlockSpec((1, pk * k3), lambda i: (0, 0)),
            pl.BlockSpec((pk * k3, pk * hid), lambda i: (0, 0)),
            pl.BlockSpec((1, pk * hid), lambda i: (0, 0)),
        ],
        out_specs=pl.BlockSpec((br, pk * hid), lambda i: (i, 0)),
        out_shape=jax.ShapeDtypeStruct((e // pk, pk * hid), jnp.bfloat16),
    )(ap, wsb, bsb, w1b, b1b)
    return tp.reshape(e, hid)


def _tc_msg(t, xg, w2p, b2p, gmat):
    e, hid = t.shape
    nf = 32
    kk = nf * nf
    be = 4000
    def body(t_ref, x_ref, w2_ref, b2_ref, g_ref, o_ref):
        we = jnp.dot(t_ref[...], w2_ref[...], preferred_element_type=jnp.float32)
        we = (we + b2_ref[...]).astype(jnp.bfloat16)
        x = x_ref[:, :nf]
        xt = jnp.tile(x.astype(jnp.bfloat16), (1, nf))
        m = xt * we
        res = jnp.dot(m, g_ref[...], preferred_element_type=jnp.float32)
        o_ref[...] = jnp.concatenate(
            [res, jnp.ones((be, 16), jnp.float32)], axis=1)

    return pl.pallas_call(
        body,
        grid=(e // be,),
        in_specs=[
            pl.BlockSpec((be, hid), lambda i: (i, 0)),
            pl.BlockSpec((be, 128), lambda i: (i, 0)),
            pl.BlockSpec((hid, kk), lambda i: (0, 0)),
            pl.BlockSpec((1, kk), lambda i: (0, 0)),
            pl.BlockSpec((kk, nf), lambda i: (0, 0)),
        ],
        out_specs=pl.BlockSpec((be, nf + 16), lambda i: (i, 0)),
        out_shape=jax.ShapeDtypeStruct((e, nf + 16), jnp.float32),
    )(t, xg, w2p, b2p.reshape(1, kk), gmat)


def kernel(h, edge_index, edge_weight, edge_attr, data,
           W0, b0, Ws, bs, W1, b1, W2, b2, Wroot, bconv):
    nf = W0.shape[1]
    src = edge_index[0].astype(jnp.int32)
    dst = edge_index[1].astype(jnp.int32)

    hid = W1.shape[1]
    w2p = W2.reshape(hid, nf, nf).transpose(0, 2, 1).reshape(hid, nf * nf)
    w2p = w2p.astype(jnp.bfloat16)
    b2p = b2.reshape(nf, nf).T.reshape(nf * nf)
    gmat = jnp.repeat(jnp.eye(nf, dtype=jnp.bfloat16), nf, axis=0)

    out0, z0 = _tc_node_proj_z(h, W0, b0, Wroot, bconv)
    t = _tc_edge_mlp(edge_attr, Ws, bs, W1, b1)

    xg1 = _sc_gather(out0, src)
    msg1 = _tc_msg(t, xg1, w2p, b2p, gmat)
    out1 = _sc_scatter_update(msg1, dst, z0)

    z1 = _tc_z(out1, Wroot, bconv)
    xg2 = _sc_gather(out1, src)
    msg2 = _tc_msg(t, xg2, w2p, b2p, gmat)
    out2 = _sc_scatter_update(msg2, dst, z1)
    return out2


# speedup vs baseline: 1.2261x; 1.1459x over previous
import functools

import jax
import jax.numpy as jnp
from jax import lax
from jax.experimental import pallas as pl
from jax.experimental.pallas import tpu as pltpu
from jax.experimental.pallas import tpu_sc as plsc

NC = 2
NS = 16
NW = NC * NS
CH = 1000


_SC_PARAMS = pltpu.CompilerParams(use_tc_tiling_on_sc=False)


def _mesh():
    return plsc.VectorSubcoreMesh(core_axis_name="c", subcore_axis_name="s",
                                  num_cores=NC, num_subcores=NS)


def _sc_gather(table, idx):
    n, f = table.shape
    e = idx.shape[0]
    per_w = e // NW
    nch = per_w // CH

    @functools.partial(
        pl.kernel,
        out_type=jax.ShapeDtypeStruct((e, 128), jnp.float32),
        mesh=_mesh(),
        compiler_params=_SC_PARAMS,
        scratch_types=[
            pltpu.VMEM((CH,), jnp.int32),
            pltpu.VMEM((CH, f), jnp.float32),
            pltpu.SemaphoreType.DMA,
        ],
    )
    def gk(table_hbm, idx_hbm, out_hbm, idx_v, rows_v, sem):
        wid = lax.axis_index("s") * NC + lax.axis_index("c")
        base = wid * per_w
        for k in range(nch):
            off = base + k * CH
            pltpu.sync_copy(idx_hbm.at[pl.ds(off, CH)], idx_v)
            pltpu.async_copy(table_hbm.at[idx_v], rows_v, sem).wait()
            pltpu.sync_copy(rows_v, out_hbm.at[pl.ds(off, CH), pl.ds(0, f)])

    return gk(table, idx)


def _sc_scatter_update(msg48, dst, z):
    e = dst.shape[0]
    n = z.shape[0]
    nl = n // NC
    ntr = (nl + NS) // NS
    npad = ntr * NS
    per_t = e // NS
    ch = 1200
    nch = per_t // ch
    tail = per_t - nch * ch
    last = nl - (NS - 1) * ntr
    zeros = jnp.zeros((npad, 48), jnp.float32)

    @functools.partial(
        pl.kernel,
        out_type=jax.ShapeDtypeStruct((n, 32), jnp.float32),
        mesh=_mesh(),
        compiler_params=_SC_PARAMS,
        scratch_types=[
            pltpu.VMEM((ch, 48), jnp.float32),
            pltpu.VMEM((ch,), jnp.int32),
            pltpu.VMEM((tail, 48), jnp.float32),
            pltpu.VMEM((tail,), jnp.int32),
            pltpu.VMEM((ntr, 48), jnp.float32),
            pltpu.VMEM((ntr, 32), jnp.float32),
            pltpu.VMEM_SHARED((npad, 48), jnp.float32),
        ],
    )
    def sk(msg_hbm, dst_hbm, z_hbm, zeros_hbm, out_hbm,
           msg_v, idx_v, msgt_v, idxt_v, acc_t, z_t, acc_sh):
        c = lax.axis_index("c")
        s = lax.axis_index("s")
        lo = c * nl
        pltpu.sync_copy(zeros_hbm.at[pl.ds(s * ntr, ntr)],
                        acc_sh.at[pl.ds(s * ntr, ntr)])
        plsc.subcore_barrier()

        def remap(idx_ref, nvec):
            def one(v, carry):
                vec = idx_ref[pl.ds(v * 16, 16)]
                loc = vec - lo
                ok = (loc >= 0) & (loc < nl)
                trash = nl + (vec & 7)
                idx_ref[pl.ds(v * 16, 16)] = jnp.where(ok, loc, trash)
                return carry
            lax.fori_loop(0, nvec, one, 0)

        base = s * per_t
        for k in range(nch):
            off = base + k * ch
            pltpu.sync_copy(dst_hbm.at[pl.ds(off, ch)], idx_v)
            pltpu.sync_copy(msg_hbm.at[pl.ds(off, ch), pl.ds(0, 48)], msg_v)
            remap(idx_v, ch // 16)
            pltpu.sync_copy(msg_v, acc_sh.at[idx_v], add=True)
        off = base + nch * ch
        pltpu.sync_copy(dst_hbm.at[pl.ds(off, tail)], idxt_v)
        pltpu.sync_copy(msg_hbm.at[pl.ds(off, tail), pl.ds(0, 48)], msgt_v)
        remap(idxt_v, tail // 16)
        pltpu.sync_copy(msgt_v, acc_sh.at[idxt_v], add=True)
        plsc.subcore_barrier()

        row0 = s * ntr
        grow = lo + row0
        pltpu.sync_copy(acc_sh.at[pl.ds(row0, ntr)], acc_t)

        def upd(r, carry):
            degv = acc_t[r, pl.ds(32, 16)]
            dinv = 1.0 / jnp.maximum(degv, 1.0)
            for h in range(2):
                a = acc_t[r, pl.ds(h * 16, 16)]
                zz = z_t[r, pl.ds(h * 16, 16)]
                z_t[r, pl.ds(h * 16, 16)] = jnp.maximum(a * dinv + zz, 0.0)
            return carry

        @pl.when(s < NS - 1)
        def _():
            pltpu.sync_copy(z_hbm.at[pl.ds(grow, ntr)], z_t)
            lax.fori_loop(0, ntr, upd, 0)
            pltpu.sync_copy(z_t, out_hbm.at[pl.ds(grow, ntr)])

        @pl.when(s == NS - 1)
        def _():
            pltpu.sync_copy(z_hbm.at[pl.ds(grow, last)], z_t.at[pl.ds(0, last)])
            lax.fori_loop(0, last, upd, 0)
            pltpu.sync_copy(z_t.at[pl.ds(0, last)], out_hbm.at[pl.ds(grow, last)])

    return sk(msg48, dst, z, zeros)


def _tc_node_proj_z(h, w0, b0, wroot, bconv):
    n, d = h.shape
    f = w0.shape[1]
    bn = 2000

    def body(h_ref, w_ref, b_ref, wr_ref, bc_ref, o_ref, z_ref):
        acc = jnp.dot(h_ref[...], w_ref[...], preferred_element_type=jnp.float32)
        o = jnp.maximum(acc + b_ref[...], 0.0)
        o_ref[...] = o
        z_ref[...] = jnp.dot(o, wr_ref[...],
                             preferred_element_type=jnp.float32) + bc_ref[...]

    return pl.pallas_call(
        body,
        grid=(n // bn,),
        in_specs=[
            pl.BlockSpec((bn, d), lambda i: (i, 0)),
            pl.BlockSpec((d, f), lambda i: (0, 0)),
            pl.BlockSpec((1, f), lambda i: (0, 0)),
            pl.BlockSpec((f, f), lambda i: (0, 0)),
            pl.BlockSpec((1, f), lambda i: (0, 0)),
        ],
        out_specs=[
            pl.BlockSpec((bn, f), lambda i: (i, 0)),
            pl.BlockSpec((bn, f), lambda i: (i, 0)),
        ],
        out_shape=[
            jax.ShapeDtypeStruct((n, f), jnp.float32),
            jax.ShapeDtypeStruct((n, f), jnp.float32),
        ],
    )(h, w0, b0.reshape(1, f), wroot, bconv.reshape(1, f))


def _tc_z(out, wroot, bconv):
    n, f = out.shape
    bn = 2000

    def body(o_ref, wr_ref, bc_ref, z_ref):
        z_ref[...] = jnp.dot(o_ref[...], wr_ref[...],
                             preferred_element_type=jnp.float32) + bc_ref[...]

    return pl.pallas_call(
        body,
        grid=(n // bn,),
        in_specs=[
            pl.BlockSpec((bn, f), lambda i: (i, 0)),
            pl.BlockSpec((f, f), lambda i: (0, 0)),
            pl.BlockSpec((1, f), lambda i: (0, 0)),
        ],
        out_specs=pl.BlockSpec((bn, f), lambda i: (i, 0)),
        out_shape=jax.ShapeDtypeStruct((n, f), jnp.float32),
    )(out, wroot, bconv.reshape(1, f))


def _tc_edge_mlp(edge_attr, ws, bs, w1, b1):
    e, ng = edge_attr.shape
    k3 = ws.shape[1]
    hid = w1.shape[1]
    pk = 128 // ng
    ap = edge_attr.reshape(e // pk, pk * ng)
    wsb = jnp.kron(jnp.eye(pk, dtype=jnp.float32), ws)
    bsb = jnp.tile(bs, pk).reshape(1, pk * k3)
    w1b = jnp.kron(jnp.eye(pk, dtype=jnp.float32), w1)
    b1b = jnp.tile(b1, pk).reshape(1, pk * hid)
    br = 400

    def body(a_ref, ws_ref, bs_ref, w1_ref, b1_ref, o_ref):
        ea = jnp.dot(a_ref[...], ws_ref[...], preferred_element_type=jnp.float32)
        ea = jnp.maximum(ea + bs_ref[...], 0.0)
        t = jnp.dot(ea, w1_ref[...], preferred_element_type=jnp.float32)
        o_ref[...] = jnp.maximum(t + b1_ref[...], 0.0)

    tp = pl.pallas_call(
        body,
        grid=(e // pk // br,),
        in_specs=[
            pl.BlockSpec((br, pk * ng), lambda i: (i, 0)),
            pl.BlockSpec((pk * ng, pk * k3), lambda i: (0, 0)),
            pl.BlockSpec((1, pk * k3), lambda i: (0, 0)),
            pl.BlockSpec((pk * k3, pk * hid), lambda i: (0, 0)),
            pl.BlockSpec((1, pk * hid), lambda i: (0, 0)),
        ],
        out_specs=pl.BlockSpec((br, pk * hid), lambda i: (i, 0)),
        out_shape=jax.ShapeDtypeStruct((e // pk, pk * hid), jnp.float32),
    )(ap, wsb, bsb, w1b, b1b)
    return tp.reshape(e, hid)


def _tc_msg(t, xg, w2p, b2p, gmat):
    e, hid = t.shape
    nf = 32
    kk = nf * nf
    be = 4000
    def body(t_ref, x_ref, w2_ref, b2_ref, g_ref, o_ref):
        tb = t_ref[...].astype(jnp.bfloat16)
        we = jnp.dot(tb, w2_ref[...], preferred_element_type=jnp.float32)
        we = (we + b2_ref[...]).astype(jnp.bfloat16)
        x = x_ref[:, :nf]
        xt = jnp.tile(x.astype(jnp.bfloat16), (1, nf))
        m = xt * we
        res = jnp.dot(m, g_ref[...], preferred_element_type=jnp.float32)
        o_ref[:, :48] = jnp.concatenate(
            [res, jnp.ones((be, 16), jnp.float32)], axis=1)

    return pl.pallas_call(
        body,
        grid=(e // be,),
        in_specs=[
            pl.BlockSpec((be, hid), lambda i: (i, 0)),
            pl.BlockSpec((be, 128), lambda i: (i, 0)),
            pl.BlockSpec((hid, kk), lambda i: (0, 0)),
            pl.BlockSpec((1, kk), lambda i: (0, 0)),
            pl.BlockSpec((kk, nf), lambda i: (0, 0)),
        ],
        out_specs=pl.BlockSpec((be, 128), lambda i: (i, 0)),
        out_shape=jax.ShapeDtypeStruct((e, 128), jnp.float32),
    )(t, xg, w2p, b2p.reshape(1, kk), gmat)


def kernel(h, edge_index, edge_weight, edge_attr, data,
           W0, b0, Ws, bs, W1, b1, W2, b2, Wroot, bconv):
    nf = W0.shape[1]
    src = edge_index[0].astype(jnp.int32)
    dst = edge_index[1].astype(jnp.int32)

    hid = W1.shape[1]
    w2p = W2.reshape(hid, nf, nf).transpose(0, 2, 1).reshape(hid, nf * nf)
    w2p = w2p.astype(jnp.bfloat16)
    b2p = b2.reshape(nf, nf).T.reshape(nf * nf)
    gmat = jnp.repeat(jnp.eye(nf, dtype=jnp.bfloat16), nf, axis=0)

    out0, z0 = _tc_node_proj_z(h, W0, b0, Wroot, bconv)
    t = _tc_edge_mlp(edge_attr, Ws, bs, W1, b1)

    xg1 = _sc_gather(out0, src)
    msg1 = _tc_msg(t, xg1, w2p, b2p, gmat)
    out1 = _sc_scatter_update(msg1, dst, z0)

    z1 = _tc_z(out1, Wroot, bconv)
    xg2 = _sc_gather(out1, src)
    msg2 = _tc_msg(t, xg2, w2p, b2p, gmat)
    out2 = _sc_scatter_update(msg2, dst, z1)
    return out2


# speedup vs baseline: 1.4754x; 1.2033x over previous
import functools

import jax
import jax.numpy as jnp
from jax import lax
from jax.experimental import pallas as pl
from jax.experimental.pallas import tpu as pltpu
from jax.experimental.pallas import tpu_sc as plsc

NC = 2
NS = 16
NW = NC * NS
CH = 1000


_SC_PARAMS = pltpu.CompilerParams(use_tc_tiling_on_sc=False)


def _mesh():
    return plsc.VectorSubcoreMesh(core_axis_name="c", subcore_axis_name="s",
                                  num_cores=NC, num_subcores=NS)


def _sc_gather(table, idx):
    n, f = table.shape
    e = idx.shape[0]
    per_w = e // NW
    nch = per_w // CH

    @functools.partial(
        pl.kernel,
        out_type=jax.ShapeDtypeStruct((e, 128), jnp.float32),
        mesh=_mesh(),
        compiler_params=_SC_PARAMS,
        scratch_types=[
            pltpu.VMEM((CH,), jnp.int32),
            pltpu.VMEM((CH, f), jnp.float32),
            pltpu.SemaphoreType.DMA,
        ],
    )
    def gk(table_hbm, idx_hbm, out_hbm, idx_v, rows_v, sem):
        wid = lax.axis_index("s") * NC + lax.axis_index("c")
        base = wid * per_w
        for k in range(nch):
            off = base + k * CH
            pltpu.sync_copy(idx_hbm.at[pl.ds(off, CH)], idx_v)
            pltpu.async_copy(table_hbm.at[idx_v], rows_v, sem).wait()
            pltpu.sync_copy(rows_v, out_hbm.at[pl.ds(off, CH), pl.ds(0, f)])

    return gk(table, idx)


def _sc_scatter_update(msg48, dst, z):
    e = dst.shape[0]
    n = z.shape[0]
    nl = n // NC
    ntr = (nl + NS) // NS
    npad = ntr * NS
    per_t = e // NS
    ch = 1200
    nch = per_t // ch
    tail = per_t - nch * ch
    last = nl - (NS - 1) * ntr
    zeros = jnp.zeros((npad, 48), jnp.float32)

    @functools.partial(
        pl.kernel,
        out_type=jax.ShapeDtypeStruct((n, 32), jnp.float32),
        mesh=_mesh(),
        compiler_params=_SC_PARAMS,
        scratch_types=[
            pltpu.VMEM((ch, 48), jnp.float32),
            pltpu.VMEM((ch,), jnp.int32),
            pltpu.VMEM((tail, 48), jnp.float32),
            pltpu.VMEM((tail,), jnp.int32),
            pltpu.VMEM((ntr, 48), jnp.float32),
            pltpu.VMEM((ntr, 32), jnp.float32),
            pltpu.VMEM_SHARED((npad, 48), jnp.float32),
        ],
    )
    def sk(msg_hbm, dst_hbm, z_hbm, zeros_hbm, out_hbm,
           msg_v, idx_v, msgt_v, idxt_v, acc_t, z_t, acc_sh):
        c = lax.axis_index("c")
        s = lax.axis_index("s")
        lo = c * nl
        pltpu.sync_copy(zeros_hbm.at[pl.ds(s * ntr, ntr)],
                        acc_sh.at[pl.ds(s * ntr, ntr)])
        plsc.subcore_barrier()

        def remap(idx_ref, nvec):
            def one(v, carry):
                vec = idx_ref[pl.ds(v * 16, 16)]
                loc = vec - lo
                ok = (loc >= 0) & (loc < nl)
                trash = nl + (vec & 7)
                idx_ref[pl.ds(v * 16, 16)] = jnp.where(ok, loc, trash)
                return carry
            lax.fori_loop(0, nvec, one, 0)

        base = s * per_t
        for k in range(nch):
            off = base + k * ch
            pltpu.sync_copy(dst_hbm.at[pl.ds(off, ch)], idx_v)
            pltpu.sync_copy(msg_hbm.at[pl.ds(off, ch), pl.ds(0, 48)], msg_v)
            remap(idx_v, ch // 16)
            pltpu.sync_copy(msg_v, acc_sh.at[idx_v], add=True)
        off = base + nch * ch
        pltpu.sync_copy(dst_hbm.at[pl.ds(off, tail)], idxt_v)
        pltpu.sync_copy(msg_hbm.at[pl.ds(off, tail), pl.ds(0, 48)], msgt_v)
        remap(idxt_v, tail // 16)
        pltpu.sync_copy(msgt_v, acc_sh.at[idxt_v], add=True)
        plsc.subcore_barrier()

        row0 = s * ntr
        grow = lo + row0
        pltpu.sync_copy(acc_sh.at[pl.ds(row0, ntr)], acc_t)

        def upd(r, carry):
            degv = acc_t[r, pl.ds(32, 16)]
            dinv = 1.0 / jnp.maximum(degv, 1.0)
            for h in range(2):
                a = acc_t[r, pl.ds(h * 16, 16)]
                zz = z_t[r, pl.ds(h * 16, 16)]
                z_t[r, pl.ds(h * 16, 16)] = jnp.maximum(a * dinv + zz, 0.0)
            return carry

        @pl.when(s < NS - 1)
        def _():
            pltpu.sync_copy(z_hbm.at[pl.ds(grow, ntr)], z_t)
            lax.fori_loop(0, ntr, upd, 0)
            pltpu.sync_copy(z_t, out_hbm.at[pl.ds(grow, ntr)])

        @pl.when(s == NS - 1)
        def _():
            pltpu.sync_copy(z_hbm.at[pl.ds(grow, last)], z_t.at[pl.ds(0, last)])
            lax.fori_loop(0, last, upd, 0)
            pltpu.sync_copy(z_t.at[pl.ds(0, last)], out_hbm.at[pl.ds(grow, last)])

    return sk(msg48, dst, z, zeros)


def _tc_node_proj_z(h, w0, b0, wroot, bconv):
    n, d = h.shape
    f = w0.shape[1]
    bn = 2000

    def body(h_ref, w_ref, b_ref, wr_ref, bc_ref, o_ref, z_ref):
        acc = jnp.dot(h_ref[...], w_ref[...], preferred_element_type=jnp.float32)
        o = jnp.maximum(acc + b_ref[...], 0.0)
        o_ref[...] = o
        z_ref[...] = jnp.dot(o, wr_ref[...],
                             preferred_element_type=jnp.float32) + bc_ref[...]

    return pl.pallas_call(
        body,
        grid=(n // bn,),
        in_specs=[
            pl.BlockSpec((bn, d), lambda i: (i, 0)),
            pl.BlockSpec((d, f), lambda i: (0, 0)),
            pl.BlockSpec((1, f), lambda i: (0, 0)),
            pl.BlockSpec((f, f), lambda i: (0, 0)),
            pl.BlockSpec((1, f), lambda i: (0, 0)),
        ],
        out_specs=[
            pl.BlockSpec((bn, f), lambda i: (i, 0)),
            pl.BlockSpec((bn, f), lambda i: (i, 0)),
        ],
        out_shape=[
            jax.ShapeDtypeStruct((n, f), jnp.float32),
            jax.ShapeDtypeStruct((n, f), jnp.float32),
        ],
    )(h, w0, b0.reshape(1, f), wroot, bconv.reshape(1, f))


def _tc_z(out, wroot, bconv):
    n, f = out.shape
    bn = 2000

    def body(o_ref, wr_ref, bc_ref, z_ref):
        z_ref[...] = jnp.dot(o_ref[...], wr_ref[...],
                             preferred_element_type=jnp.float32) + bc_ref[...]

    return pl.pallas_call(
        body,
        grid=(n // bn,),
        in_specs=[
            pl.BlockSpec((bn, f), lambda i: (i, 0)),
            pl.BlockSpec((f, f), lambda i: (0, 0)),
            pl.BlockSpec((1, f), lambda i: (0, 0)),
        ],
        out_specs=pl.BlockSpec((bn, f), lambda i: (i, 0)),
        out_shape=jax.ShapeDtypeStruct((n, f), jnp.float32),
    )(out, wroot, bconv.reshape(1, f))


def _tc_edge_mlp(edge_attr, ws, bs, w1, b1):
    e, ng = edge_attr.shape
    k3 = ws.shape[1]
    hid = w1.shape[1]
    at = edge_attr.T
    be = 3200

    def body(a_ref, ws_ref, bs_ref, w1_ref, b1_ref, o_ref):
        ea = lax.dot_general(a_ref[...], ws_ref[...],
                             (((0,), (0,)), ((), ())),
                             preferred_element_type=jnp.float32)
        ea = jnp.maximum(ea + bs_ref[...], 0.0)
        t = jnp.dot(ea, w1_ref[...], preferred_element_type=jnp.float32)
        o_ref[...] = jnp.maximum(t + b1_ref[...], 0.0)

    return pl.pallas_call(
        body,
        grid=(e // be,),
        in_specs=[
            pl.BlockSpec((ng, be), lambda i: (0, i)),
            pl.BlockSpec((ng, k3), lambda i: (0, 0)),
            pl.BlockSpec((1, k3), lambda i: (0, 0)),
            pl.BlockSpec((k3, hid), lambda i: (0, 0)),
            pl.BlockSpec((1, hid), lambda i: (0, 0)),
        ],
        out_specs=pl.BlockSpec((be, hid), lambda i: (i, 0)),
        out_shape=jax.ShapeDtypeStruct((e, hid), jnp.float32),
    )(at, ws, bs.reshape(1, k3), w1, b1.reshape(1, hid))


def _tc_msg(t, xg, w2p, b2p, gmat):
    e, hid = t.shape
    nf = 32
    kk = nf * nf
    be = 4000
    def body(t_ref, x_ref, w2_ref, b2_ref, g_ref, o_ref):
        tb = t_ref[...].astype(jnp.bfloat16)
        we = jnp.dot(tb, w2_ref[...], preferred_element_type=jnp.float32)
        we = (we + b2_ref[...]).astype(jnp.bfloat16)
        x = x_ref[:, :nf]
        xt = jnp.tile(x.astype(jnp.bfloat16), (1, nf))
        m = xt * we
        res = jnp.dot(m, g_ref[...], preferred_element_type=jnp.float32)
        o_ref[:, :48] = jnp.concatenate(
            [res, jnp.ones((be, 16), jnp.float32)], axis=1)

    return pl.pallas_call(
        body,
        grid=(e // be,),
        in_specs=[
            pl.BlockSpec((be, hid), lambda i: (i, 0)),
            pl.BlockSpec((be, 128), lambda i: (i, 0)),
            pl.BlockSpec((hid, kk), lambda i: (0, 0)),
            pl.BlockSpec((1, kk), lambda i: (0, 0)),
            pl.BlockSpec((kk, nf), lambda i: (0, 0)),
        ],
        out_specs=pl.BlockSpec((be, 128), lambda i: (i, 0)),
        out_shape=jax.ShapeDtypeStruct((e, 128), jnp.float32),
    )(t, xg, w2p, b2p.reshape(1, kk), gmat)


def kernel(h, edge_index, edge_weight, edge_attr, data,
           W0, b0, Ws, bs, W1, b1, W2, b2, Wroot, bconv):
    nf = W0.shape[1]
    src = edge_index[0].astype(jnp.int32)
    dst = edge_index[1].astype(jnp.int32)

    hid = W1.shape[1]
    w2p = W2.reshape(hid, nf, nf).transpose(0, 2, 1).reshape(hid, nf * nf)
    w2p = w2p.astype(jnp.bfloat16)
    b2p = b2.reshape(nf, nf).T.reshape(nf * nf)
    gmat = jnp.repeat(jnp.eye(nf, dtype=jnp.bfloat16), nf, axis=0)

    out0, z0 = _tc_node_proj_z(h, W0, b0, Wroot, bconv)
    t = _tc_edge_mlp(edge_attr, Ws, bs, W1, b1)

    xg1 = _sc_gather(out0, src)
    msg1 = _tc_msg(t, xg1, w2p, b2p, gmat)
    out1 = _sc_scatter_update(msg1, dst, z0)

    z1 = _tc_z(out1, Wroot, bconv)
    xg2 = _sc_gather(out1, src)
    msg2 = _tc_msg(t, xg2, w2p, b2p, gmat)
    out2 = _sc_scatter_update(msg2, dst, z1)
    return out2
